# baseline (speedup 1.0000x reference)
import jax
import jax.numpy as jnp
from jax.experimental import pallas as pl

B, D, T = 16, 64, 2048
K = 1024
N = B * T
TB = 512
NB = N // TB


def _rvq_kernel(x_ref, cb0_ref, cb1_ref, cb2_ref, idx_ref, loss_ref):
    step = pl.program_id(0)
    xb = x_ref[0]
    z = xb.T

    total = jnp.float32(0.0)
    idx = None
    for q, cb_ref in enumerate((cb0_ref, cb1_ref, cb2_ref)):
        cb = cb_ref[...]
        zz = jnp.sum(z * z, axis=1, keepdims=True)
        cc = jnp.sum(cb * cb, axis=1)[None, :]
        m = jax.lax.dot_general(z, cb, (((1,), (1,)), ((), ())),
                                preferred_element_type=jnp.float32)
        d = zz - 2.0 * m + cc
        dmin = jnp.min(d, axis=1, keepdims=True)
        iota = jax.lax.broadcasted_iota(jnp.int32, d.shape, 1)
        idx = jnp.min(jnp.where(d == dmin, iota, K), axis=1)
        total = total + jnp.sum(dmin)
        if q < 2:
            onehot = (iota == idx[:, None]).astype(jnp.float32)
            codes = jax.lax.dot_general(
                onehot, cb, (((1,), (0,)), ((), ())),
                precision=jax.lax.Precision.HIGHEST,
                preferred_element_type=jnp.float32)
            z = z - codes

    idx_ref[0, 0, :] = idx

    @pl.when(step == 0)
    def _():
        loss_ref[:, :] = jnp.zeros((1, 1), jnp.float32)

    loss_ref[:, :] += total


def kernel(x, cb0, cb1, cb2):
    nT = T // TB

    idx_blocks, loss_sum = pl.pallas_call(
        _rvq_kernel,
        grid=(NB,),
        in_specs=[
            pl.BlockSpec((1, D, TB), lambda i: (i // nT, 0, i % nT)),
            pl.BlockSpec((K, D), lambda i: (0, 0)),
            pl.BlockSpec((K, D), lambda i: (0, 0)),
            pl.BlockSpec((K, D), lambda i: (0, 0)),
        ],
        out_specs=[
            pl.BlockSpec((1, 1, TB), lambda i: (i, 0, 0)),
            pl.BlockSpec((1, 1), lambda i: (0, 0)),
        ],
        out_shape=[
            jax.ShapeDtypeStruct((NB, 1, TB), jnp.int32),
            jax.ShapeDtypeStruct((1, 1), jnp.float32),
        ],
    )(x, cb0, cb1, cb2)

    code_index = idx_blocks.reshape(B, T)
    loss = (loss_sum[0, 0] / jnp.float32(N * D)).astype(jnp.float32)
    loss = loss.reshape(())
    return (code_index, loss, loss)


# speedup vs baseline: 1.4467x; 1.4467x over previous
import jax
import jax.numpy as jnp
from jax.experimental import pallas as pl

B, D, T = 16, 64, 2048
K = 1024
N = B * T
TB = 512
NB = N // TB


def _rvq_kernel(x_ref, cb0_ref, cb1_ref, cb2_ref,
                h0_ref, m0_ref, l0_ref, h1_ref, m1_ref, l1_ref,
                idx_ref, loss_ref):
    step = pl.program_id(0)
    xb = x_ref[0]
    z = xb.T

    total = jnp.float32(0.0)
    idx = None
    for q, cb_ref in enumerate((cb0_ref, cb1_ref, cb2_ref)):
        cb = cb_ref[...]
        zz = jnp.sum(z * z, axis=1, keepdims=True)
        cc = jnp.sum(cb * cb, axis=1)[None, :]
        m = jax.lax.dot_general(z, cb, (((1,), (1,)), ((), ())),
                                preferred_element_type=jnp.float32)
        d = zz - 2.0 * m + cc
        dmin = jnp.min(d, axis=1, keepdims=True)
        iota = jax.lax.broadcasted_iota(jnp.int32, d.shape, 1)
        idx = jnp.min(jnp.where(d == dmin, iota, K), axis=1)
        total = total + jnp.sum(dmin)
        if q < 2:
            h, m_, l = ((h0_ref, m0_ref, l0_ref),
                        (h1_ref, m1_ref, l1_ref))[q]
            onehot = (iota == idx[:, None]).astype(jnp.bfloat16)
            dims = (((1,), (0,)), ((), ()))
            codes = (jax.lax.dot_general(onehot, h[...], dims,
                                         preferred_element_type=jnp.float32)
                     + jax.lax.dot_general(onehot, m_[...], dims,
                                           preferred_element_type=jnp.float32)
                     + jax.lax.dot_general(onehot, l[...], dims,
                                           preferred_element_type=jnp.float32))
            z = z - codes

    idx_ref[0, 0, :] = idx

    @pl.when(step == 0)
    def _():
        loss_ref[:, :] = jnp.zeros((1, 1), jnp.float32)

    loss_ref[:, :] += total


def _split3(cb):
    hi = cb.astype(jnp.bfloat16)
    r1 = cb - hi.astype(jnp.float32)
    mid = r1.astype(jnp.bfloat16)
    lo = (r1 - mid.astype(jnp.float32)).astype(jnp.bfloat16)
    return hi, mid, lo


def kernel(x, cb0, cb1, cb2):
    nT = T // TB

    idx_blocks, loss_sum = pl.pallas_call(
        _rvq_kernel,
        grid=(NB,),
        in_specs=[
            pl.BlockSpec((1, D, TB), lambda i: (i // nT, 0, i % nT)),
            pl.BlockSpec((K, D), lambda i: (0, 0)),
            pl.BlockSpec((K, D), lambda i: (0, 0)),
            pl.BlockSpec((K, D), lambda i: (0, 0)),
        ] + [pl.BlockSpec((K, D), lambda i: (0, 0)) for _ in range(6)],
        out_specs=[
            pl.BlockSpec((1, 1, TB), lambda i: (i, 0, 0)),
            pl.BlockSpec((1, 1), lambda i: (0, 0)),
        ],
        out_shape=[
            jax.ShapeDtypeStruct((NB, 1, TB), jnp.int32),
            jax.ShapeDtypeStruct((1, 1), jnp.float32),
        ],
    )(x, cb0, cb1, cb2, *_split3(cb0), *_split3(cb1))

    code_index = idx_blocks.reshape(B, T)
    loss = (loss_sum[0, 0] / jnp.float32(N * D)).astype(jnp.float32)
    loss = loss.reshape(())
    return (code_index, loss, loss)


# speedup vs baseline: 1.6044x; 1.1090x over previous
import jax
import jax.numpy as jnp
from jax.experimental import pallas as pl

B, D, T = 16, 64, 2048
K = 1024
N = B * T
TB = 256
NB = N // TB


def _rvq_kernel(x_ref, cb0_ref, cb1_ref, cb2_ref,
                cc0_ref, cc1_ref, cc2_ref,
                h0_ref, m0_ref, l0_ref, h1_ref, m1_ref, l1_ref,
                idx_ref, loss_ref):
    step = pl.program_id(0)
    xb = x_ref[0]
    z = xb.T

    total = jnp.float32(0.0)
    idx = None
    for q, (cb_ref, cc_ref) in enumerate(((cb0_ref, cc0_ref),
                                          (cb1_ref, cc1_ref),
                                          (cb2_ref, cc2_ref))):
        cb = cb_ref[...]
        zz = jnp.sum(z * z, axis=1, keepdims=True)
        cc = cc_ref[...]
        m = jax.lax.dot_general(z, cb, (((1,), (1,)), ((), ())),
                                preferred_element_type=jnp.float32)
        d = zz - 2.0 * m + cc
        dmin = jnp.min(d, axis=1, keepdims=True)
        iota = jax.lax.broadcasted_iota(jnp.int32, d.shape, 1)
        idx = jnp.argmin(d, axis=1)
        total = total + jnp.sum(dmin)
        if q < 2:
            h, m_, l = ((h0_ref, m0_ref, l0_ref),
                        (h1_ref, m1_ref, l1_ref))[q]
            onehot = (iota == idx[:, None]).astype(jnp.bfloat16)
            dims = (((1,), (0,)), ((), ()))
            codes = (jax.lax.dot_general(onehot, h[...], dims,
                                         preferred_element_type=jnp.float32)
                     + jax.lax.dot_general(onehot, m_[...], dims,
                                           preferred_element_type=jnp.float32)
                     + jax.lax.dot_general(onehot, l[...], dims,
                                           preferred_element_type=jnp.float32))
            z = z - codes

    idx_ref[0, 0, :] = idx

    @pl.when(step == 0)
    def _():
        loss_ref[:, :] = jnp.zeros((1, 1), jnp.float32)

    loss_ref[:, :] += total


def _split3(cb):
    hi = cb.astype(jnp.bfloat16)
    r1 = cb - hi.astype(jnp.float32)
    mid = r1.astype(jnp.bfloat16)
    lo = (r1 - mid.astype(jnp.float32)).astype(jnp.bfloat16)
    return hi, mid, lo


def kernel(x, cb0, cb1, cb2):
    nT = T // TB

    idx_blocks, loss_sum = pl.pallas_call(
        _rvq_kernel,
        grid=(NB,),
        in_specs=[
            pl.BlockSpec((1, D, TB), lambda i: (i // nT, 0, i % nT)),
            pl.BlockSpec((K, D), lambda i: (0, 0)),
            pl.BlockSpec((K, D), lambda i: (0, 0)),
            pl.BlockSpec((K, D), lambda i: (0, 0)),
        ] + [pl.BlockSpec((1, K), lambda i: (0, 0)) for _ in range(3)]
          + [pl.BlockSpec((K, D), lambda i: (0, 0)) for _ in range(6)],
        out_specs=[
            pl.BlockSpec((1, 1, TB), lambda i: (i, 0, 0)),
            pl.BlockSpec((1, 1), lambda i: (0, 0)),
        ],
        out_shape=[
            jax.ShapeDtypeStruct((NB, 1, TB), jnp.int32),
            jax.ShapeDtypeStruct((1, 1), jnp.float32),
        ],
    )(x, cb0, cb1, cb2,
      jnp.sum(cb0 * cb0, axis=1)[None, :],
      jnp.sum(cb1 * cb1, axis=1)[None, :],
      jnp.sum(cb2 * cb2, axis=1)[None, :],
      *_split3(cb0), *_split3(cb1))

    code_index = idx_blocks.reshape(B, T)
    loss = (loss_sum[0, 0] / jnp.float32(N * D)).astype(jnp.float32)
    loss = loss.reshape(())
    return (code_index, loss, loss)
